# baseline (speedup 1.0000x reference)
import functools

import jax
import jax.numpy as jnp
from jax import lax
from jax.experimental import pallas as pl
from jax.experimental.pallas import tpu as pltpu
from jax.experimental.pallas import tpu_sc as plsc

N = 10000
E = 320000
D = 128
HID = 64
HEADS = 4
OUT = 32
NEG_SLOPE = 0.2
LN_EPS = 1e-5

NC = 2
NS = 16
NW = NC * NS
CH = 128
NCHK = 82
E1 = E + N
PT = NCHK * CH
EPAD = PT * NW
NACC = 12032
STR = NACC // NS
BE = 2048
BN = 1000
W128 = 128

_f32 = jnp.float32


def _make_gather():
    mesh = plsc.VectorSubcoreMesh(core_axis_name="c", subcore_axis_name="s")

    @functools.partial(
        pl.kernel,
        mesh=mesh,
        out_type=(
            jax.ShapeDtypeStruct((EPAD, W128), _f32),
            jax.ShapeDtypeStruct((EPAD, W128), _f32),
        ),
        scratch_types=[
            pltpu.VMEM((NCHK, CH), jnp.int32),
            pltpu.VMEM((NCHK, CH), jnp.int32),
            pltpu.VMEM((2, CH, W128), _f32),
            pltpu.VMEM((2, CH, W128), _f32),
        ] + [pltpu.SemaphoreType.DMA] * 8,
    )
    def gather_k(tab, src3, dst3, xj_out, xi_out,
                 idxs, idxd, xj_v, xi_v,
                 ga0, ga1, gb0, gb1, wa0, wa1, wb0, wb1):
        wid = lax.axis_index("s") * NC + lax.axis_index("c")
        pltpu.sync_copy(src3.at[wid], idxs)
        pltpu.sync_copy(dst3.at[wid], idxd)
        gsem = (ga0, ga1)
        hsem = (gb0, gb1)
        wsem = (wa0, wa1)
        vsem = (wb0, wb1)

        def fire_gather(k, b):
            pltpu.async_copy(tab.at[idxs.at[k]], xj_v.at[b], gsem[b])
            pltpu.async_copy(tab.at[idxd.at[k]], xi_v.at[b], hsem[b])

        def wait_gather(b):
            pltpu.make_async_copy(tab.at[idxs.at[0]], xj_v.at[b],
                                  gsem[b]).wait()
            pltpu.make_async_copy(tab.at[idxd.at[0]], xi_v.at[b],
                                  hsem[b]).wait()

        def fire_write(k, b):
            base = wid * PT + k * CH
            pltpu.async_copy(xj_v.at[b], xj_out.at[pl.ds(base, CH)], wsem[b])
            pltpu.async_copy(xi_v.at[b], xi_out.at[pl.ds(base, CH)], vsem[b])

        def wait_write(b):
            pltpu.make_async_copy(xj_v.at[b], xj_out.at[pl.ds(0, CH)],
                                  wsem[b]).wait()
            pltpu.make_async_copy(xi_v.at[b], xi_out.at[pl.ds(0, CH)],
                                  vsem[b]).wait()

        for b in (0, 1):
            fire_gather(b, b)

        def body(g, carry):
            k = 2 * g
            for b in (0, 1):
                wait_gather(b)
                fire_write(k + b, b)
            for b in (0, 1):
                wait_write(b)
                fire_gather(k + 2 + b, b)
            return carry

        lax.fori_loop(0, NCHK // 2 - 1, body, 0)
        for b in (0, 1):
            wait_gather(b)
            fire_write(NCHK - 2 + b, b)
        for b in (0, 1):
            wait_write(b)

    return gather_k


def _make_scatter():
    mesh = plsc.VectorSubcoreMesh(core_axis_name="c", subcore_axis_name="s")

    @functools.partial(
        pl.kernel,
        mesh=mesh,
        out_type=jax.ShapeDtypeStruct((NC, NACC, W128), _f32),
        scratch_types=[
            pltpu.VMEM((2, CH), jnp.int32),
            pltpu.VMEM((2, CH, W128), _f32),
            pltpu.VMEM_SHARED((NACC, W128), _f32),
        ] + [pltpu.SemaphoreType.DMA] * 4,
    )
    def scatter_k(rows_hbm, dst3, zero_hbm, out_acc,
                  idxd, rows_v, acc, l0, l1, i0, i1):
        cid = lax.axis_index("c")
        sid = lax.axis_index("s")
        wid = sid * NC + cid
        r0 = sid * STR
        pltpu.sync_copy(zero_hbm.at[pl.ds(r0, STR)], acc.at[pl.ds(r0, STR)])
        plsc.subcore_barrier()
        lsem = (l0, l1)
        isem = (i0, i1)

        def fire_load(k, b):
            base = wid * PT + k * CH
            pltpu.async_copy(dst3.at[wid, k], idxd.at[b], isem[b])
            pltpu.async_copy(rows_hbm.at[pl.ds(base, CH)], rows_v.at[b],
                             lsem[b])

        def wait_load(b):
            pltpu.make_async_copy(dst3.at[0, 0], idxd.at[b], isem[b]).wait()
            pltpu.make_async_copy(rows_hbm.at[pl.ds(0, CH)], rows_v.at[b],
                                  lsem[b]).wait()

        for b in (0, 1):
            fire_load(b, b)

        def body(g, carry):
            k = 2 * g
            for b in (0, 1):
                wait_load(b)
                pltpu.sync_copy(rows_v.at[b], acc.at[idxd.at[b]],
                                add=True)
                fire_load(k + 2 + b, b)
            return carry

        lax.fori_loop(0, NCHK // 2 - 1, body, 0)
        for b in (0, 1):
            wait_load(b)
            pltpu.sync_copy(rows_v.at[b], acc.at[idxd.at[b]],
                            add=True)
        plsc.subcore_barrier()
        pltpu.sync_copy(acc.at[pl.ds(r0, STR)], out_acc.at[cid, pl.ds(r0, STR)])

    return scatter_k


def _dot(a, b):
    return jnp.dot(a, b, preferred_element_type=_f32)


def _proj1_body(x_ref, wcat_ref, acat_ref, tab_ref, p_ref):
    t = _dot(x_ref[...], wcat_ref[...])
    tab_ref[...] = t
    p_ref[...] = _dot(jnp.abs(t), acat_ref[...])


def _proj1(x, wcat, acat):
    return pl.pallas_call(
        _proj1_body,
        grid=(N // BN,),
        in_specs=[
            pl.BlockSpec((BN, D), lambda i: (i, 0)),
            pl.BlockSpec((D, W128), lambda i: (0, 0)),
            pl.BlockSpec((W128, 16), lambda i: (0, 0)),
        ],
        out_specs=[
            pl.BlockSpec((BN, W128), lambda i: (i, 0)),
            pl.BlockSpec((BN, 16), lambda i: (i, 0)),
        ],
        out_shape=[
            jax.ShapeDtypeStruct((N, W128), _f32),
            jax.ShapeDtypeStruct((N, 16), _f32),
        ],
    )(x, wcat, acat)


def _edge_body(xj_ref, xi_ref, asig_ref, r128_ref, q128_ref, b_ref, rv_ref,
               *, C):
    xj = xj_ref[...]
    t = xj[:, 0:C] + xi_ref[:, C:2 * C]
    s = jnp.where(t > 0, t, NEG_SLOPE * t)
    alpha = _dot(s, asig_ref[...])
    e = jnp.exp(alpha - b_ref[...])
    rv_ref[...] = xj * _dot(e, r128_ref[...]) + _dot(e, q128_ref[...])


def _edge(xj, xi, asig, r128, q128, bvec, C):
    return pl.pallas_call(
        functools.partial(_edge_body, C=C),
        grid=(EPAD // BE,),
        in_specs=[
            pl.BlockSpec((BE, W128), lambda i: (i, 0)),
            pl.BlockSpec((BE, W128), lambda i: (i, 0)),
            pl.BlockSpec((C, 4), lambda i: (0, 0)),
            pl.BlockSpec((4, W128), lambda i: (0, 0)),
            pl.BlockSpec((4, W128), lambda i: (0, 0)),
            pl.BlockSpec((1, 4), lambda i: (0, 0)),
        ],
        out_specs=pl.BlockSpec((BE, W128), lambda i: (i, 0)),
        out_shape=jax.ShapeDtypeStruct((EPAD, W128), _f32),
    )(xj, xi, asig, r128, q128, bvec)


def _mid_body(acc_ref, rrep_ref, b1_ref, g1_ref, be1_ref, w2cat_ref,
              a2cat_ref, tab2_ref, p2_ref):
    num = acc_ref[0, :, 0:HID] + acc_ref[1, :, 0:HID]
    den = acc_ref[0, :, HID:HID + 4] + acc_ref[1, :, HID:HID + 4]
    h = num / (_dot(den, rrep_ref[...]) + 1e-16) + b1_ref[...]
    mu = jnp.mean(h, axis=-1, keepdims=True)
    dlt = h - mu
    var = jnp.mean(dlt * dlt, axis=-1, keepdims=True)
    hn = dlt * lax.rsqrt(var + LN_EPS) * g1_ref[...] + be1_ref[...]
    he = jnp.where(hn > 0, hn, jnp.exp(hn) - 1.0)
    t2 = _dot(he, w2cat_ref[...])
    tab2_ref[...] = t2
    p2_ref[...] = _dot(jnp.abs(t2), a2cat_ref[...])


def _mid(acc, rrep, b1, g1, be1, w2cat, a2cat):
    return pl.pallas_call(
        _mid_body,
        grid=(N // BN,),
        in_specs=[
            pl.BlockSpec((NC, BN, W128), lambda i: (0, i, 0)),
            pl.BlockSpec((4, HID), lambda i: (0, 0)),
            pl.BlockSpec((1, HID), lambda i: (0, 0)),
            pl.BlockSpec((1, HID), lambda i: (0, 0)),
            pl.BlockSpec((1, HID), lambda i: (0, 0)),
            pl.BlockSpec((HID, W128), lambda i: (0, 0)),
            pl.BlockSpec((W128, 16), lambda i: (0, 0)),
        ],
        out_specs=[
            pl.BlockSpec((BN, W128), lambda i: (i, 0)),
            pl.BlockSpec((BN, 16), lambda i: (i, 0)),
        ],
        out_shape=[
            jax.ShapeDtypeStruct((N, W128), _f32),
            jax.ShapeDtypeStruct((N, 16), _f32),
        ],
    )(acc, rrep, b1, g1, be1, w2cat, a2cat)


def _final_body(acc_ref, b2_ref, g2_ref, be2_ref, out_ref):
    num = acc_ref[0, :, 0:OUT] + acc_ref[1, :, 0:OUT]
    den = acc_ref[0, :, OUT:OUT + 1] + acc_ref[1, :, OUT:OUT + 1]
    h = num / (den + 1e-16) + b2_ref[...]
    mu = jnp.mean(h, axis=-1, keepdims=True)
    dlt = h - mu
    var = jnp.mean(dlt * dlt, axis=-1, keepdims=True)
    out_ref[...] = dlt * lax.rsqrt(var + LN_EPS) * g2_ref[...] + be2_ref[...]


def _final(acc, b2, g2, be2):
    return pl.pallas_call(
        _final_body,
        grid=(N // BN,),
        in_specs=[
            pl.BlockSpec((NC, BN, W128), lambda i: (0, i, 0)),
            pl.BlockSpec((1, OUT), lambda i: (0, 0)),
            pl.BlockSpec((1, OUT), lambda i: (0, 0)),
            pl.BlockSpec((1, OUT), lambda i: (0, 0)),
        ],
        out_specs=pl.BlockSpec((BN, OUT), lambda i: (i, 0)),
        out_shape=jax.ShapeDtypeStruct((N, OUT), _f32),
    )(acc, b2, g2, be2)


def kernel(x, edge_index, Wl1, Wr1, att1, b1, g1, be1,
           Wl2, Wr2, att2, b2, g2, be2):
    f32 = _f32
    loop = jnp.arange(N, dtype=jnp.int32)
    pad = EPAD - E1
    src = jnp.concatenate([edge_index[0], loop,
                           jnp.zeros((pad,), jnp.int32)])
    dst = jnp.concatenate([edge_index[1], loop,
                           jnp.full((pad,), N, jnp.int32)])
    src3 = src.reshape(NW, NCHK, CH)
    dst3 = dst.reshape(NW, NCHK, CH)

    mask = jnp.repeat(jnp.eye(HEADS, dtype=f32), HID // HEADS, axis=0)
    aabs1 = mask * jnp.abs(att1).reshape(HID, 1)
    z64_8 = jnp.zeros((HID, 8), f32)
    z64_4 = jnp.zeros((HID, 4), f32)
    acat1 = jnp.concatenate([
        jnp.concatenate([aabs1, z64_4, z64_8], axis=1),
        jnp.concatenate([z64_4, aabs1, z64_8], axis=1),
    ], axis=0)
    w1cat = jnp.concatenate([Wl1, Wr1], axis=1)
    asig1 = mask * att1.reshape(HID, 1)
    r128_1 = jnp.concatenate([mask.T, jnp.zeros((4, HID), f32)], axis=1)
    q128_1 = jnp.zeros((4, W128), f32).at[
        jnp.arange(4), HID + jnp.arange(4)].set(1.0)
    rrep1 = mask.T

    a2abs = jnp.abs(att2).reshape(OUT, 1)
    z32 = jnp.zeros((OUT, 1), f32)
    acat2 = jnp.concatenate([
        jnp.concatenate([a2abs] + [z32] * 15, axis=1),
        jnp.concatenate([z32] * 4 + [a2abs] + [z32] * 11, axis=1),
        jnp.zeros((HID, 16), f32),
    ], axis=0)
    w2cat = jnp.concatenate([Wl2, Wr2, jnp.zeros((HID, HID), f32)], axis=1)
    asig2 = jnp.concatenate([att2.reshape(OUT, 1),
                             jnp.zeros((OUT, 3), f32)], axis=1)
    r128_2 = jnp.zeros((4, W128), f32).at[0, 0:OUT].set(1.0)
    q128_2 = jnp.zeros((4, W128), f32).at[0, OUT].set(1.0)

    zacc = jnp.zeros((NACC, W128), f32)
    gather = _make_gather()
    scatter = _make_scatter()

    tab1, p1 = _proj1(x, w1cat, acat1)
    bvec1 = (jnp.max(p1[:, 0:4], axis=0) + jnp.max(p1[:, 4:8], axis=0)
             ).reshape(1, 4)
    xj1, xi1 = gather(tab1, src3, dst3)
    rv1 = _edge(xj1, xi1, asig1, r128_1, q128_1, bvec1, HID)
    acc1 = scatter(rv1, dst3, zacc)
    tab2, p2 = _mid(acc1, rrep1, b1.reshape(1, HID),
                    g1.reshape(1, HID), be1.reshape(1, HID), w2cat, acat2)

    b2s = jnp.max(p2[:, 0], axis=0) + jnp.max(p2[:, 4], axis=0)
    bvec2 = jnp.concatenate([b2s.reshape(1, 1), jnp.zeros((1, 3), f32)], 1)
    xj2, xi2 = gather(tab2, src3, dst3)
    rv2 = _edge(xj2, xi2, asig2, r128_2, q128_2, bvec2, OUT)
    acc2 = scatter(rv2, dst3, zacc)
    return _final(acc2, b2.reshape(1, OUT), g2.reshape(1, OUT),
                  be2.reshape(1, OUT))


# speedup vs baseline: 1.0287x; 1.0287x over previous
import functools

import jax
import jax.numpy as jnp
from jax import lax
from jax.experimental import pallas as pl
from jax.experimental.pallas import tpu as pltpu
from jax.experimental.pallas import tpu_sc as plsc

N = 10000
E = 320000
D = 128
HID = 64
HEADS = 4
OUT = 32
NEG_SLOPE = 0.2
LN_EPS = 1e-5

NC = 2
NS = 16
NW = NC * NS
CH = 128
E1 = E + N
PT = 10368
EPAD = PT * NW
NACC = 12032
STR = NACC // NS
BE = 2048
BN = 1000
W128 = 128

_f32 = jnp.float32


def _make_gather():
    nch = PT // CH
    mesh = plsc.VectorSubcoreMesh(core_axis_name="c", subcore_axis_name="s")

    @functools.partial(
        pl.kernel,
        mesh=mesh,
        out_type=(
            jax.ShapeDtypeStruct((EPAD, W128), _f32),
            jax.ShapeDtypeStruct((EPAD, W128), _f32),
        ),
        scratch_types=[
            pltpu.VMEM((CH,), jnp.int32),
            pltpu.VMEM((CH,), jnp.int32),
            pltpu.VMEM((CH, W128), _f32),
            pltpu.VMEM((CH, W128), _f32),
            pltpu.SemaphoreType.DMA,
            pltpu.SemaphoreType.DMA,
        ],
    )
    def gather_k(tab_hbm, src_hbm, dst_hbm, xj_out, xi_out,
                 idxs_v, idxd_v, xj_v, xi_v, sem_a, sem_b):
        wid = lax.axis_index("s") * NC + lax.axis_index("c")

        def body(k, carry):
            base = wid * PT + k * CH
            pltpu.sync_copy(src_hbm.at[pl.ds(base, CH)], idxs_v)
            pltpu.sync_copy(dst_hbm.at[pl.ds(base, CH)], idxd_v)
            cp_a = pltpu.async_copy(tab_hbm.at[idxs_v], xj_v, sem_a)
            cp_b = pltpu.async_copy(tab_hbm.at[idxd_v], xi_v, sem_b)
            cp_a.wait()
            cp_b.wait()
            pltpu.sync_copy(xj_v, xj_out.at[pl.ds(base, CH)])
            pltpu.sync_copy(xi_v, xi_out.at[pl.ds(base, CH)])
            return carry

        lax.fori_loop(0, nch, body, 0)

    return gather_k


def _make_scatter():
    nch = PT // CH
    mesh = plsc.VectorSubcoreMesh(core_axis_name="c", subcore_axis_name="s")

    @functools.partial(
        pl.kernel,
        mesh=mesh,
        out_type=jax.ShapeDtypeStruct((NC, NACC, W128), _f32),
        scratch_types=[
            pltpu.VMEM((CH,), jnp.int32),
            pltpu.VMEM((CH, W128), _f32),
            pltpu.VMEM_SHARED((NACC, W128), _f32),
        ],
    )
    def scatter_k(rows_hbm, dst_hbm, zero_hbm, out_acc, idx_v, rows_v, acc):
        cid = lax.axis_index("c")
        sid = lax.axis_index("s")
        wid = sid * NC + cid
        r0 = sid * STR
        pltpu.sync_copy(zero_hbm.at[pl.ds(r0, STR)], acc.at[pl.ds(r0, STR)])
        plsc.subcore_barrier()

        def body(k, carry):
            base = wid * PT + k * CH
            pltpu.sync_copy(dst_hbm.at[pl.ds(base, CH)], idx_v)
            pltpu.sync_copy(rows_hbm.at[pl.ds(base, CH)], rows_v)
            pltpu.sync_copy(rows_v, acc.at[idx_v], add=True)
            return carry

        lax.fori_loop(0, nch, body, 0)
        plsc.subcore_barrier()
        pltpu.sync_copy(acc.at[pl.ds(r0, STR)], out_acc.at[cid, pl.ds(r0, STR)])

    return scatter_k


def _dot(a, b):
    return jnp.dot(a, b, preferred_element_type=_f32)


def _proj1_body(x_ref, wcat_ref, acat_ref, tab_ref, p_ref):
    t = _dot(x_ref[...], wcat_ref[...])
    tab_ref[...] = t
    p_ref[...] = _dot(jnp.abs(t), acat_ref[...])


def _proj1(x, wcat, acat):
    return pl.pallas_call(
        _proj1_body,
        grid=(N // BN,),
        in_specs=[
            pl.BlockSpec((BN, D), lambda i: (i, 0)),
            pl.BlockSpec((D, W128), lambda i: (0, 0)),
            pl.BlockSpec((W128, 16), lambda i: (0, 0)),
        ],
        out_specs=[
            pl.BlockSpec((BN, W128), lambda i: (i, 0)),
            pl.BlockSpec((BN, 16), lambda i: (i, 0)),
        ],
        out_shape=[
            jax.ShapeDtypeStruct((N, W128), _f32),
            jax.ShapeDtypeStruct((N, 16), _f32),
        ],
    )(x, wcat, acat)


def _edge_body(xj_ref, xi_ref, asig_ref, r128_ref, q128_ref, b_ref, rv_ref,
               *, C):
    xj = xj_ref[...]
    t = xj[:, 0:C] + xi_ref[:, C:2 * C]
    s = jnp.where(t > 0, t, NEG_SLOPE * t)
    alpha = _dot(s, asig_ref[...])
    e = jnp.exp(alpha - b_ref[...])
    rv_ref[...] = xj * _dot(e, r128_ref[...]) + _dot(e, q128_ref[...])


def _edge(xj, xi, asig, r128, q128, bvec, C):
    return pl.pallas_call(
        functools.partial(_edge_body, C=C),
        grid=(EPAD // BE,),
        in_specs=[
            pl.BlockSpec((BE, W128), lambda i: (i, 0)),
            pl.BlockSpec((BE, W128), lambda i: (i, 0)),
            pl.BlockSpec((C, 4), lambda i: (0, 0)),
            pl.BlockSpec((4, W128), lambda i: (0, 0)),
            pl.BlockSpec((4, W128), lambda i: (0, 0)),
            pl.BlockSpec((1, 4), lambda i: (0, 0)),
        ],
        out_specs=pl.BlockSpec((BE, W128), lambda i: (i, 0)),
        out_shape=jax.ShapeDtypeStruct((EPAD, W128), _f32),
    )(xj, xi, asig, r128, q128, bvec)


def _mid_body(acc_ref, rrep_ref, b1_ref, g1_ref, be1_ref, w2cat_ref,
              a2cat_ref, tab2_ref, p2_ref):
    num = acc_ref[0, :, 0:HID] + acc_ref[1, :, 0:HID]
    den = acc_ref[0, :, HID:HID + 4] + acc_ref[1, :, HID:HID + 4]
    h = num / (_dot(den, rrep_ref[...]) + 1e-16) + b1_ref[...]
    mu = jnp.mean(h, axis=-1, keepdims=True)
    dlt = h - mu
    var = jnp.mean(dlt * dlt, axis=-1, keepdims=True)
    hn = dlt * lax.rsqrt(var + LN_EPS) * g1_ref[...] + be1_ref[...]
    he = jnp.where(hn > 0, hn, jnp.exp(hn) - 1.0)
    t2 = _dot(he, w2cat_ref[...])
    tab2_ref[...] = t2
    p2_ref[...] = _dot(jnp.abs(t2), a2cat_ref[...])


def _mid(acc, rrep, b1, g1, be1, w2cat, a2cat):
    return pl.pallas_call(
        _mid_body,
        grid=(N // BN,),
        in_specs=[
            pl.BlockSpec((NC, BN, W128), lambda i: (0, i, 0)),
            pl.BlockSpec((4, HID), lambda i: (0, 0)),
            pl.BlockSpec((1, HID), lambda i: (0, 0)),
            pl.BlockSpec((1, HID), lambda i: (0, 0)),
            pl.BlockSpec((1, HID), lambda i: (0, 0)),
            pl.BlockSpec((HID, W128), lambda i: (0, 0)),
            pl.BlockSpec((W128, 16), lambda i: (0, 0)),
        ],
        out_specs=[
            pl.BlockSpec((BN, W128), lambda i: (i, 0)),
            pl.BlockSpec((BN, 16), lambda i: (i, 0)),
        ],
        out_shape=[
            jax.ShapeDtypeStruct((N, W128), _f32),
            jax.ShapeDtypeStruct((N, 16), _f32),
        ],
    )(acc, rrep, b1, g1, be1, w2cat, a2cat)


def _final_body(acc_ref, b2_ref, g2_ref, be2_ref, out_ref):
    num = acc_ref[0, :, 0:OUT] + acc_ref[1, :, 0:OUT]
    den = acc_ref[0, :, OUT:OUT + 1] + acc_ref[1, :, OUT:OUT + 1]
    h = num / (den + 1e-16) + b2_ref[...]
    mu = jnp.mean(h, axis=-1, keepdims=True)
    dlt = h - mu
    var = jnp.mean(dlt * dlt, axis=-1, keepdims=True)
    out_ref[...] = dlt * lax.rsqrt(var + LN_EPS) * g2_ref[...] + be2_ref[...]


def _final(acc, b2, g2, be2):
    return pl.pallas_call(
        _final_body,
        grid=(N // BN,),
        in_specs=[
            pl.BlockSpec((NC, BN, W128), lambda i: (0, i, 0)),
            pl.BlockSpec((1, OUT), lambda i: (0, 0)),
            pl.BlockSpec((1, OUT), lambda i: (0, 0)),
            pl.BlockSpec((1, OUT), lambda i: (0, 0)),
        ],
        out_specs=pl.BlockSpec((BN, OUT), lambda i: (i, 0)),
        out_shape=jax.ShapeDtypeStruct((N, OUT), _f32),
    )(acc, b2, g2, be2)


def kernel(x, edge_index, Wl1, Wr1, att1, b1, g1, be1,
           Wl2, Wr2, att2, b2, g2, be2):
    f32 = _f32
    loop = jnp.arange(N, dtype=jnp.int32)
    pad = EPAD - E1
    src = jnp.concatenate([edge_index[0], loop,
                           jnp.zeros((pad,), jnp.int32)])
    dst = jnp.concatenate([edge_index[1], loop,
                           jnp.full((pad,), N, jnp.int32)])

    mask = jnp.repeat(jnp.eye(HEADS, dtype=f32), HID // HEADS, axis=0)
    aabs1 = mask * jnp.abs(att1).reshape(HID, 1)
    z64_8 = jnp.zeros((HID, 8), f32)
    z64_4 = jnp.zeros((HID, 4), f32)
    acat1 = jnp.concatenate([
        jnp.concatenate([aabs1, z64_4, z64_8], axis=1),
        jnp.concatenate([z64_4, aabs1, z64_8], axis=1),
    ], axis=0)
    w1cat = jnp.concatenate([Wl1, Wr1], axis=1)
    asig1 = mask * att1.reshape(HID, 1)
    r128_1 = jnp.concatenate([mask.T, jnp.zeros((4, HID), f32)], axis=1)
    q128_1 = jnp.zeros((4, W128), f32).at[
        jnp.arange(4), HID + jnp.arange(4)].set(1.0)
    rrep1 = mask.T

    a2abs = jnp.abs(att2).reshape(OUT, 1)
    z32 = jnp.zeros((OUT, 1), f32)
    acat2 = jnp.concatenate([
        jnp.concatenate([a2abs] + [z32] * 15, axis=1),
        jnp.concatenate([z32] * 4 + [a2abs] + [z32] * 11, axis=1),
        jnp.zeros((HID, 16), f32),
    ], axis=0)
    w2cat = jnp.concatenate([Wl2, Wr2, jnp.zeros((HID, HID), f32)], axis=1)
    asig2 = jnp.concatenate([att2.reshape(OUT, 1),
                             jnp.zeros((OUT, 3), f32)], axis=1)
    r128_2 = jnp.zeros((4, W128), f32).at[0, 0:OUT].set(1.0)
    q128_2 = jnp.zeros((4, W128), f32).at[0, OUT].set(1.0)

    zacc = jnp.zeros((NACC, W128), f32)
    gather = _make_gather()
    scatter = _make_scatter()

    tab1, p1 = _proj1(x, w1cat, acat1)
    bvec1 = (jnp.max(p1[:, 0:4], axis=0) + jnp.max(p1[:, 4:8], axis=0)
             ).reshape(1, 4)
    xj1, xi1 = gather(tab1, src, dst)
    rv1 = _edge(xj1, xi1, asig1, r128_1, q128_1, bvec1, HID)
    acc1 = scatter(rv1, dst, zacc)
    tab2, p2 = _mid(acc1, rrep1, b1.reshape(1, HID),
                    g1.reshape(1, HID), be1.reshape(1, HID), w2cat, acat2)

    b2s = jnp.max(p2[:, 0], axis=0) + jnp.max(p2[:, 4], axis=0)
    bvec2 = jnp.concatenate([b2s.reshape(1, 1), jnp.zeros((1, 3), f32)], 1)
    xj2, xi2 = gather(tab2, src, dst)
    rv2 = _edge(xj2, xi2, asig2, r128_2, q128_2, bvec2, OUT)
    acc2 = scatter(rv2, dst, zacc)
    return _final(acc2, b2.reshape(1, OUT), g2.reshape(1, OUT),
                  be2.reshape(1, OUT))
